# baseline (speedup 1.0000x reference)
import functools

import jax
import jax.numpy as jnp
from jax import lax
from jax.experimental import pallas as pl
from jax.experimental.pallas import tpu as pltpu
from jax.experimental.pallas import tpu_sc as plsc

B = 16384
D = 32
NC = 2
NS = 16
NW = NC * NS
BPW = B // NW
L = 16


def _sc_stage(user_idx, book_idx, user_embedding, user_bias_flat,
              book_embedding, book_bias_flat):
    mesh = plsc.VectorSubcoreMesh(core_axis_name="c", subcore_axis_name="s")

    @functools.partial(
        pl.kernel,
        out_type=(
            jax.ShapeDtypeStruct((NW * L,), jnp.float32),
            jax.ShapeDtypeStruct((B,), jnp.float32),
        ),
        mesh=mesh,
        compiler_params=pltpu.CompilerParams(use_tc_tiling_on_sc=False),
        scratch_types=[
            pltpu.VMEM((BPW,), jnp.int32),
            pltpu.VMEM((BPW,), jnp.int32),
            pltpu.VMEM((BPW, D), jnp.float32),
            pltpu.VMEM((BPW, D), jnp.float32),
            pltpu.VMEM((BPW,), jnp.float32),
            pltpu.VMEM((BPW,), jnp.float32),
            pltpu.VMEM((BPW,), jnp.float32),
            pltpu.VMEM((L,), jnp.float32),
            pltpu.SemaphoreType.DMA,
            pltpu.SemaphoreType.DMA,
            pltpu.SemaphoreType.DMA,
            pltpu.SemaphoreType.DMA,
        ],
    )
    def k(uidx_hbm, bidx_hbm, uemb_hbm, ubias_hbm, bemb_hbm, bbias_hbm,
          part_hbm, t_hbm,
          uidx_v, bidx_v, urows_v, brows_v, ub_v, bb_v, t_v, acc_v,
          sem_u, sem_b, sem_ub, sem_bb):
        wid = lax.axis_index("s") * NC + lax.axis_index("c")
        base = wid * BPW
        pltpu.sync_copy(uidx_hbm.at[pl.ds(base, BPW)], uidx_v)
        pltpu.sync_copy(bidx_hbm.at[pl.ds(base, BPW)], bidx_v)
        cu = pltpu.async_copy(uemb_hbm.at[uidx_v], urows_v, sem_u)
        cb = pltpu.async_copy(bemb_hbm.at[bidx_v], brows_v, sem_b)
        cub = pltpu.async_copy(ubias_hbm.at[uidx_v], ub_v, sem_ub)
        cbb = pltpu.async_copy(bbias_hbm.at[bidx_v], bb_v, sem_bb)

        cub.wait()
        cbb.wait()
        for j in range(BPW // L):
            sl = pl.ds(j * L, L)
            t_v[sl] = ub_v[sl] + bb_v[sl]
        pltpu.sync_copy(t_v, t_hbm.at[pl.ds(base, BPW)])

        cu.wait()
        cb.wait()

        def body(i, acc):
            a0 = urows_v[i, pl.ds(0, L)] * brows_v[i, pl.ds(0, L)]
            a1 = urows_v[i, pl.ds(L, L)] * brows_v[i, pl.ds(L, L)]
            return acc + a0 + a1

        acc = lax.fori_loop(0, BPW, body, jnp.zeros((L,), jnp.float32))
        acc_v[...] = acc
        pltpu.sync_copy(acc_v, part_hbm.at[pl.ds(wid * L, L)])

    return k(user_idx, book_idx, user_embedding, user_bias_flat,
             book_embedding, book_bias_flat)


def _tc_finish(partials, t):
    def body(p_ref, t_ref, o_ref):
        s = jnp.sum(p_ref[...])
        o_ref[...] = jax.nn.sigmoid(t_ref[...] + s)

    out = pl.pallas_call(
        body,
        out_shape=jax.ShapeDtypeStruct((128, 128), jnp.float32),
    )(partials.reshape(4, 128), t.reshape(128, 128))
    return out.reshape(B, 1)


def kernel(inputs, user_embedding, user_bias, book_embedding, book_bias):
    user_idx = inputs[:, 0].astype(jnp.int32)
    book_idx = inputs[:, 1].astype(jnp.int32)
    partials, t = _sc_stage(
        user_idx, book_idx,
        user_embedding, user_bias.reshape(-1),
        book_embedding, book_bias.reshape(-1),
    )
    return _tc_finish(partials, t)
